# baseline (speedup 1.0000x reference)
import functools

import jax
import jax.numpy as jnp
from jax import lax
from jax.experimental import pallas as pl
from jax.experimental.pallas import tpu as pltpu
from jax.experimental.pallas import tpu_sc as plsc

N = 10000
E = 320000
D = 128

NC, NS = 2, 16
NW = NC * NS
EPT = E // NW
C = 96
ITERS = 105
CHUNKS = ITERS
EPTP = CHUNKS * C
NP = 10240
RPT = NP // NS


def _sc_aggregate(x, srcp, dstp):
    mesh = plsc.VectorSubcoreMesh(core_axis_name="c", subcore_axis_name="s")

    @functools.partial(
        pl.kernel,
        out_type=jax.ShapeDtypeStruct((NC, NP, D), jnp.float32),
        mesh=mesh,
        scratch_types=[
            pltpu.VMEM_SHARED((NP, D), jnp.float32),
            pltpu.VMEM((CHUNKS * C,), jnp.int32),
            pltpu.VMEM((CHUNKS, C), jnp.int32),
            pltpu.VMEM((C, D), jnp.float32),
            pltpu.VMEM((C, D), jnp.float32),
            pltpu.SemaphoreType.DMA,
            pltpu.SemaphoreType.DMA,
        ],
    )
    def agg_kernel(x_hbm, src_hbm, dst_hbm, parts_hbm,
                   acc, src_v, dst_v, rows0, rows1, g0, g1):
        c = lax.axis_index("c")
        s = lax.axis_index("s")
        g = c * NS + s
        rows = (rows0, rows1)
        gsem = (g0, g1)

        zv = jnp.zeros((16,), jnp.float32)

        def zstep(i, carry):
            rows0[i // 8, pl.ds((i % 8) * 16, 16)] = zv
            return carry

        lax.fori_loop(0, C * 8, zstep, 0)

        WC = 64

        def zfan(k, carry):
            pltpu.async_copy(rows0.at[pl.ds(0, WC)],
                             acc.at[pl.ds(s * RPT + k * WC, WC)], g0)
            return carry

        def zdrain(k, carry):
            pltpu.make_async_copy(rows0.at[pl.ds(0, WC)],
                                  acc.at[pl.ds(s * RPT + k * WC, WC)],
                                  g0).wait()
            return carry

        lax.fori_loop(0, RPT // WC, zfan, 0)
        pltpu.sync_copy(src_hbm.at[g], src_v)
        pltpu.sync_copy(dst_hbm.at[g], dst_v)
        lax.fori_loop(0, RPT // WC, zdrain, 0)
        plsc.subcore_barrier()

        def sidx(j):
            return src_v.at[pl.ds(j * C, C)]

        pltpu.async_copy(x_hbm.at[sidx(0)], rows0, g0)

        def body(jj, carry):
            for b in range(2):
                j = 2 * jj + b
                pltpu.make_async_copy(x_hbm.at[sidx(j)],
                                      rows[b], gsem[b]).wait()
                pltpu.async_copy(x_hbm.at[sidx(j + 1)],
                                 rows[1 - b], gsem[1 - b])
                pltpu.sync_copy(rows[b], acc.at[dst_v.at[j]], add=True)
            return carry

        lax.fori_loop(0, (ITERS - 1) // 2, body, 0)
        pltpu.make_async_copy(x_hbm.at[sidx(ITERS - 1)], rows0, g0).wait()
        pltpu.sync_copy(rows0, acc.at[dst_v.at[ITERS - 1]], add=True)
        plsc.subcore_barrier()

        WC2 = 64

        def wstep(k, carry):
            pltpu.sync_copy(acc.at[pl.ds(s * RPT + k * WC2, WC2)],
                            rows0.at[pl.ds(0, WC2)])
            pltpu.sync_copy(rows0.at[pl.ds(0, WC2)],
                            parts_hbm.at[c, pl.ds(s * RPT + k * WC2, WC2)])
            return carry

        lax.fori_loop(0, RPT // WC2, wstep, 0)

    return agg_kernel(x, srcp, dstp)


def _tc_body(x_ref, p_ref, w1a_ref, w1b_ref, b1_ref, w2_ref, b2_ref, o_ref):
    agg = p_ref[0] + p_ref[1]
    h = jnp.dot(agg, w1a_ref[...], preferred_element_type=jnp.float32)
    h += jnp.dot(x_ref[...], w1b_ref[...], preferred_element_type=jnp.float32)
    h = jnp.maximum(h + b1_ref[...], 0.0)
    o_ref[...] = (jnp.dot(h, w2_ref[...], preferred_element_type=jnp.float32)
                  + b2_ref[...])


def _tc_finish(x, parts, W1, b1, W2, b2):
    R = 1000
    grid = (N // R,)
    w1a = W1[:D]
    w1b = W1[D:]
    return pl.pallas_call(
        _tc_body,
        grid=grid,
        in_specs=[
            pl.BlockSpec((R, D), lambda i: (i, 0)),
            pl.BlockSpec((NC, R, D), lambda i: (0, i, 0)),
            pl.BlockSpec((D, D), lambda i: (0, 0)),
            pl.BlockSpec((D, D), lambda i: (0, 0)),
            pl.BlockSpec((1, D), lambda i: (0, 0)),
            pl.BlockSpec((D, D), lambda i: (0, 0)),
            pl.BlockSpec((1, D), lambda i: (0, 0)),
        ],
        out_specs=pl.BlockSpec((R, D), lambda i: (i, 0)),
        out_shape=jax.ShapeDtypeStruct((N, D), jnp.float32),
    )(x, parts, w1a, w1b, b1.reshape(1, D), W2, b2.reshape(1, D))


def kernel(x, edge_index, W1, b1, W2, b2):
    ei = edge_index.astype(jnp.int32)
    pad = EPTP - EPT
    srcp = jnp.pad(ei[0].reshape(NW, EPT), ((0, 0), (0, pad)))
    deadrows = (N + (jnp.arange(NW * pad, dtype=jnp.int32) % (NP - N))
                ).reshape(NW, pad)
    dstp = jnp.concatenate([ei[1].reshape(NW, EPT), deadrows],
                           axis=1).reshape(NW, CHUNKS, C)
    parts = _sc_aggregate(x, srcp, dstp)
    return _tc_finish(x, parts, W1, b1, W2, b2)


# speedup vs baseline: 1.0408x; 1.0408x over previous
import functools

import jax
import jax.numpy as jnp
from jax import lax
from jax.experimental import pallas as pl
from jax.experimental.pallas import tpu as pltpu
from jax.experimental.pallas import tpu_sc as plsc

N = 10000
E = 320000
D = 128

NC, NS = 2, 16
NW = NC * NS
EPT = E // NW
C = 64
ITERS = 157
CHUNKS = ITERS
EPTP = CHUNKS * C
NP = 10240
RPT = NP // NS


def _sc_aggregate(x, srcp, dstp):
    mesh = plsc.VectorSubcoreMesh(core_axis_name="c", subcore_axis_name="s")

    @functools.partial(
        pl.kernel,
        out_type=jax.ShapeDtypeStruct((NC, NP, D), jnp.float32),
        mesh=mesh,
        scratch_types=[
            pltpu.VMEM_SHARED((NP, D), jnp.float32),
            pltpu.VMEM((CHUNKS * C,), jnp.int32),
            pltpu.VMEM((CHUNKS, C), jnp.int32),
            pltpu.VMEM((C, D), jnp.float32),
            pltpu.VMEM((C, D), jnp.float32),
            pltpu.SemaphoreType.DMA,
            pltpu.SemaphoreType.DMA,
        ],
    )
    def agg_kernel(x_hbm, src_hbm, dst_hbm, parts_hbm,
                   acc, src_v, dst_v, rows0, rows1, g0, g1):
        c = lax.axis_index("c")
        s = lax.axis_index("s")
        g = c * NS + s
        rows = (rows0, rows1)
        gsem = (g0, g1)

        zv = jnp.zeros((16,), jnp.float32)

        def zstep(i, carry):
            rows0[i // 8, pl.ds((i % 8) * 16, 16)] = zv
            return carry

        lax.fori_loop(0, C * 8, zstep, 0)

        def zfan(k, carry):
            pltpu.async_copy(rows0, acc.at[pl.ds(s * RPT + k * C, C)], g0)
            return carry

        def zdrain(k, carry):
            pltpu.make_async_copy(rows0,
                                  acc.at[pl.ds(s * RPT + k * C, C)], g0).wait()
            return carry

        lax.fori_loop(0, RPT // C, zfan, 0)
        pltpu.sync_copy(src_hbm.at[g], src_v)
        pltpu.sync_copy(dst_hbm.at[g], dst_v)
        lax.fori_loop(0, RPT // C, zdrain, 0)
        plsc.subcore_barrier()

        def sidx(j):
            return src_v.at[pl.ds(j * C, C)]

        pltpu.async_copy(x_hbm.at[sidx(0)], rows0, g0)

        def body(jj, carry):
            for b in range(2):
                j = 2 * jj + b
                pltpu.make_async_copy(x_hbm.at[sidx(j)],
                                      rows[b], gsem[b]).wait()
                pltpu.async_copy(x_hbm.at[sidx(j + 1)],
                                 rows[1 - b], gsem[1 - b])
                pltpu.sync_copy(rows[b], acc.at[dst_v.at[j]], add=True)
            return carry

        lax.fori_loop(0, (ITERS - 1) // 2, body, 0)
        pltpu.make_async_copy(x_hbm.at[sidx(ITERS - 1)], rows0, g0).wait()
        pltpu.sync_copy(rows0, acc.at[dst_v.at[ITERS - 1]], add=True)
        plsc.subcore_barrier()

        def wstep(k, carry):
            pltpu.sync_copy(acc.at[pl.ds(s * RPT + k * C, C)], rows0)
            pltpu.sync_copy(rows0, parts_hbm.at[c, pl.ds(s * RPT + k * C, C)])
            return carry

        lax.fori_loop(0, RPT // C, wstep, 0)

    return agg_kernel(x, srcp, dstp)


def _tc_body(x_ref, p_ref, w1a_ref, w1b_ref, b1_ref, w2_ref, b2_ref, o_ref):
    agg = p_ref[0] + p_ref[1]
    h = jnp.dot(agg, w1a_ref[...], preferred_element_type=jnp.float32)
    h += jnp.dot(x_ref[...], w1b_ref[...], preferred_element_type=jnp.float32)
    h = jnp.maximum(h + b1_ref[...], 0.0)
    o_ref[...] = (jnp.dot(h, w2_ref[...], preferred_element_type=jnp.float32)
                  + b2_ref[...])


def _tc_finish(x, parts, W1, b1, W2, b2):
    R = 1000
    grid = (N // R,)
    w1a = W1[:D]
    w1b = W1[D:]
    return pl.pallas_call(
        _tc_body,
        grid=grid,
        in_specs=[
            pl.BlockSpec((R, D), lambda i: (i, 0)),
            pl.BlockSpec((NC, R, D), lambda i: (0, i, 0)),
            pl.BlockSpec((D, D), lambda i: (0, 0)),
            pl.BlockSpec((D, D), lambda i: (0, 0)),
            pl.BlockSpec((1, D), lambda i: (0, 0)),
            pl.BlockSpec((D, D), lambda i: (0, 0)),
            pl.BlockSpec((1, D), lambda i: (0, 0)),
        ],
        out_specs=pl.BlockSpec((R, D), lambda i: (i, 0)),
        out_shape=jax.ShapeDtypeStruct((N, D), jnp.float32),
    )(x, parts, w1a, w1b, b1.reshape(1, D), W2, b2.reshape(1, D))


def kernel(x, edge_index, W1, b1, W2, b2):
    ei = edge_index.astype(jnp.int32)
    pad = EPTP - EPT
    srcp = jnp.pad(ei[0].reshape(NW, EPT), ((0, 0), (0, pad)))
    deadrows = (N + (jnp.arange(NW * pad, dtype=jnp.int32) % (NP - N))
                ).reshape(NW, pad)
    dstp = jnp.concatenate([ei[1].reshape(NW, EPT), deadrows],
                           axis=1).reshape(NW, CHUNKS, C)
    parts = _sc_aggregate(x, srcp, dstp)
    return _tc_finish(x, parts, W1, b1, W2, b2)


# speedup vs baseline: 1.3965x; 1.3417x over previous
import functools

import jax
import jax.numpy as jnp
from jax import lax
from jax.experimental import pallas as pl
from jax.experimental.pallas import tpu as pltpu
from jax.experimental.pallas import tpu_sc as plsc

N = 10000
E = 320000
D = 128

NC, NS = 2, 16
NW = NC * NS
EPT = E // NW
C = 80
ITERS = 125
CHUNKS = ITERS
EPTP = CHUNKS * C
NP = 10240
RPT = NP // NS


def _sc_aggregate(x, srcp, dstp):
    mesh = plsc.VectorSubcoreMesh(core_axis_name="c", subcore_axis_name="s")

    @functools.partial(
        pl.kernel,
        out_type=jax.ShapeDtypeStruct((NC, NP, D), jnp.float32),
        mesh=mesh,
        scratch_types=[
            pltpu.VMEM_SHARED((NP, D), jnp.float32),
            pltpu.VMEM((CHUNKS * C,), jnp.int32),
            pltpu.VMEM((CHUNKS, C), jnp.int32),
            pltpu.VMEM((C, D), jnp.float32),
            pltpu.VMEM((C, D), jnp.float32),
            pltpu.SemaphoreType.DMA,
            pltpu.SemaphoreType.DMA,
        ],
    )
    def agg_kernel(x_hbm, src_hbm, dst_hbm, parts_hbm,
                   acc, src_v, dst_v, rows0, rows1, g0, g1):
        c = lax.axis_index("c")
        s = lax.axis_index("s")
        g = c * NS + s
        rows = (rows0, rows1)
        gsem = (g0, g1)

        zv = jnp.zeros((16,), jnp.float32)

        def zstep(i, carry):
            rows0[i // 8, pl.ds((i % 8) * 16, 16)] = zv
            return carry

        lax.fori_loop(0, C * 8, zstep, 0)

        def zfan(k, carry):
            pltpu.async_copy(rows0, acc.at[pl.ds(s * RPT + k * C, C)], g0)
            return carry

        def zdrain(k, carry):
            pltpu.make_async_copy(rows0,
                                  acc.at[pl.ds(s * RPT + k * C, C)], g0).wait()
            return carry

        lax.fori_loop(0, RPT // C, zfan, 0)
        pltpu.sync_copy(src_hbm.at[g], src_v)
        pltpu.sync_copy(dst_hbm.at[g], dst_v)
        lax.fori_loop(0, RPT // C, zdrain, 0)
        plsc.subcore_barrier()

        def sidx(j):
            return src_v.at[pl.ds(j * C, C)]

        pltpu.async_copy(x_hbm.at[sidx(0)], rows0, g0)

        def body(jj, carry):
            for b in range(2):
                j = 2 * jj + b
                pltpu.make_async_copy(x_hbm.at[sidx(j)],
                                      rows[b], gsem[b]).wait()
                pltpu.async_copy(x_hbm.at[sidx(j + 1)],
                                 rows[1 - b], gsem[1 - b])
                pltpu.sync_copy(rows[b], acc.at[dst_v.at[j]], add=True)
            return carry

        lax.fori_loop(0, (ITERS - 1) // 2, body, 0)
        pltpu.make_async_copy(x_hbm.at[sidx(ITERS - 1)], rows0, g0).wait()
        pltpu.sync_copy(rows0, acc.at[dst_v.at[ITERS - 1]], add=True)
        plsc.subcore_barrier()

        def wstep(k, carry):
            pltpu.sync_copy(acc.at[pl.ds(s * RPT + k * C, C)], rows0)
            pltpu.sync_copy(rows0, parts_hbm.at[c, pl.ds(s * RPT + k * C, C)])
            return carry

        lax.fori_loop(0, RPT // C, wstep, 0)

    return agg_kernel(x, srcp, dstp)


def _tc_body(x_ref, p_ref, w1a_ref, w1b_ref, b1_ref, w2_ref, b2_ref, o_ref):
    agg = p_ref[0] + p_ref[1]
    h = jnp.dot(agg, w1a_ref[...], preferred_element_type=jnp.float32)
    h += jnp.dot(x_ref[...], w1b_ref[...], preferred_element_type=jnp.float32)
    h = jnp.maximum(h + b1_ref[...], 0.0)
    o_ref[...] = (jnp.dot(h, w2_ref[...], preferred_element_type=jnp.float32)
                  + b2_ref[...])


def _tc_finish(x, parts, W1, b1, W2, b2):
    R = 1000
    grid = (N // R,)
    w1a = W1[:D]
    w1b = W1[D:]
    return pl.pallas_call(
        _tc_body,
        grid=grid,
        in_specs=[
            pl.BlockSpec((R, D), lambda i: (i, 0)),
            pl.BlockSpec((NC, R, D), lambda i: (0, i, 0)),
            pl.BlockSpec((D, D), lambda i: (0, 0)),
            pl.BlockSpec((D, D), lambda i: (0, 0)),
            pl.BlockSpec((1, D), lambda i: (0, 0)),
            pl.BlockSpec((D, D), lambda i: (0, 0)),
            pl.BlockSpec((1, D), lambda i: (0, 0)),
        ],
        out_specs=pl.BlockSpec((R, D), lambda i: (i, 0)),
        out_shape=jax.ShapeDtypeStruct((N, D), jnp.float32),
    )(x, parts, w1a, w1b, b1.reshape(1, D), W2, b2.reshape(1, D))


def kernel(x, edge_index, W1, b1, W2, b2):
    ei = edge_index.astype(jnp.int32)
    pad = EPTP - EPT
    srcp = jnp.pad(ei[0].reshape(NW, EPT), ((0, 0), (0, pad)))
    deadrows = (N + (jnp.arange(NW * pad, dtype=jnp.int32) % (NP - N))
                ).reshape(NW, pad)
    dstp = jnp.concatenate([ei[1].reshape(NW, EPT), deadrows],
                           axis=1).reshape(NW, CHUNKS, C)
    parts = _sc_aggregate(x, srcp, dstp)
    return _tc_finish(x, parts, W1, b1, W2, b2)
